# baseline (speedup 1.0000x reference)
import functools
import math

import jax
import jax.numpy as jnp
from jax.experimental import pallas as pl
from jax.experimental.pallas import tpu as pltpu

_SIZE = 32000
_PAD_ID = 0
_SMOOTHING = 0.1
_CONF = 1.0 - _SMOOTHING
_EPS = _SMOOTHING / (_SIZE - 2)
_C0 = _CONF * math.log(_CONF) + _SMOOTHING * math.log(_EPS)

_BR = 2048
_BC = 3200


def _loss_body(tgt_ref, x_ref, o_ref, *, bc):
    j = pl.program_id(1)

    @pl.when(j == 0)
    def _init():
        o_ref[...] = jnp.zeros((1, 1, 1), jnp.float32)

    xb = x_ref[...]
    t = tgt_ref[0, 0, :]
    valid = t != _PAD_ID

    cols = j * bc + jax.lax.broadcasted_iota(jnp.int32, xb.shape, 1)
    w = jnp.where(cols == t[:, None], -_CONF, -_EPS)
    per_row = jnp.sum(xb * w, axis=1)

    @pl.when(j == 0)
    def _edge():
        o_ref[...] += jnp.sum(
            jnp.where(valid, _C0 + _EPS * xb[:, 0], 0.0)).reshape(1, 1, 1)

    o_ref[...] += jnp.sum(jnp.where(valid, per_row, 0.0)).reshape(1, 1, 1)


@jax.jit
def kernel(x, tgt):
    n, size = x.shape
    nr = n // _BR
    tgt3 = tgt.astype(jnp.int32).reshape(nr, 1, _BR)
    grid = (nr, size // _BC)
    partials = pl.pallas_call(
        functools.partial(_loss_body, bc=_BC),
        grid=grid,
        in_specs=[
            pl.BlockSpec((1, 1, _BR), lambda i, j: (i, 0, 0)),
            pl.BlockSpec((_BR, _BC), lambda i, j: (i, j)),
        ],
        out_specs=pl.BlockSpec((1, 1, 1), lambda i, j: (i, 0, 0)),
        out_shape=jax.ShapeDtypeStruct((nr, 1, 1), jnp.float32),
        compiler_params=pltpu.CompilerParams(
            dimension_semantics=("parallel", "arbitrary")),
    )(tgt3, x)
    return jnp.sum(partials)


# speedup vs baseline: 1.0073x; 1.0073x over previous
import functools
import math

import jax
import jax.numpy as jnp
from jax.experimental import pallas as pl
from jax.experimental.pallas import tpu as pltpu

_SIZE = 32000
_PAD_ID = 0
_SMOOTHING = 0.1
_CONF = 1.0 - _SMOOTHING
_EPS = _SMOOTHING / (_SIZE - 2)
_C0 = _CONF * math.log(_CONF) + _SMOOTHING * math.log(_EPS)

_BR = 1024
_BC = 6400


def _loss_body(tgt_ref, x_ref, o_ref, *, bc):
    j = pl.program_id(1)

    @pl.when(j == 0)
    def _init():
        o_ref[...] = jnp.zeros((1, 1, 1), jnp.float32)

    xb = x_ref[...]
    t = tgt_ref[0, 0, :]
    valid = t != _PAD_ID

    cols = j * bc + jax.lax.broadcasted_iota(jnp.int32, xb.shape, 1)
    w = jnp.where(cols == t[:, None], -_CONF, -_EPS)
    per_row = jnp.sum(xb * w, axis=1)

    @pl.when(j == 0)
    def _edge():
        o_ref[...] += jnp.sum(
            jnp.where(valid, _C0 + _EPS * xb[:, 0], 0.0)).reshape(1, 1, 1)

    o_ref[...] += jnp.sum(jnp.where(valid, per_row, 0.0)).reshape(1, 1, 1)


@jax.jit
def kernel(x, tgt):
    n, size = x.shape
    nr = n // _BR
    tgt3 = tgt.astype(jnp.int32).reshape(nr, 1, _BR)
    grid = (nr, size // _BC)
    partials = pl.pallas_call(
        functools.partial(_loss_body, bc=_BC),
        grid=grid,
        in_specs=[
            pl.BlockSpec((1, 1, _BR), lambda i, j: (i, 0, 0)),
            pl.BlockSpec((_BR, _BC), lambda i, j: (i, j)),
        ],
        out_specs=pl.BlockSpec((1, 1, 1), lambda i, j: (i, 0, 0)),
        out_shape=jax.ShapeDtypeStruct((nr, 1, 1), jnp.float32),
        compiler_params=pltpu.CompilerParams(
            dimension_semantics=("parallel", "arbitrary")),
    )(tgt3, x)
    return jnp.sum(partials)
